# Initial kernel scaffold; baseline (speedup 1.0000x reference)
import jax
import jax.numpy as jnp
from jax.experimental import pallas as pl


def kernel(x, params):
    raise NotImplementedError("write your pallas kernel here")


# speedup vs baseline: 3.9459x; 3.9459x over previous
import functools

import jax
import jax.numpy as jnp
from jax.experimental import pallas as pl

_INTERPRET = False

_LAYER_CFG = [
    (3, 32, 8, 1, -1),
    (32, 64, 8, 2, -1),
    (64, 96, 8, 4, -1),
    (96, 128, 12, 4, 120),
    (128, 160, 12, 6, 120),
]


def _elu(x):
    return jnp.where(x > 0, x, jnp.exp(jnp.minimum(x, 0.0)) - 1.0)


def _xconv_body(pts_ref, ptsT_ref, fts_ref, *rest, N, Pb, K, D, steps, sel):
    if sel:
        seloh_ref = rest[0]
        wrefs = rest[1:17]
        out_ref = rest[17]
        rep_out_ref = rest[18]
    else:
        rep_ref = rest[0]
        wrefs = rest[1:17]
        out_ref = rest[17]
    (Wl, bl, W1, b1, W2, b2, Wx0, bx0, Wx1, bx1, Wx2, bx2,
     Wdt, bdp, Wpp, bp) = [w[...] for w in wrefs]

    pts = pts_ref[0]
    fts = fts_ref[0]
    lifted = _elu(jnp.dot(fts, Wl, preferred_element_type=jnp.float32, precision=jax.lax.Precision.HIGHEST) + bl)

    if sel:
        seloh = seloh_ref[...]
        rep = jnp.dot(seloh, pts, preferred_element_type=jnp.float32, precision=jax.lax.Precision.HIGHEST)
        rep_out_ref[0] = rep
    else:
        rep = rep_ref[0]

    ptsT = ptsT_ref[0]
    d2 = None
    for c in range(3):
        diff = rep[:, c:c + 1] - ptsT[c:c + 1, :]
        sq = diff * diff
        d2 = sq if d2 is None else d2 + sq

    G = jnp.concatenate([pts, lifted], axis=1)
    iota = jax.lax.broadcasted_iota(jnp.int32, (Pb, N), 1)
    inf = jnp.float32(jnp.inf)

    nbr = []
    for j in range(steps):
        m = jnp.min(d2, axis=1, keepdims=True)
        cand = jnp.where(d2 <= m, iota, N)
        idx = jnp.min(cand, axis=1, keepdims=True)
        oh = iota == idx
        if j >= 1 and (j - 1) % D == 0 and len(nbr) < K:
            g = jnp.dot(oh.astype(jnp.float32), G,
                        preferred_element_type=jnp.float32, precision=jax.lax.Precision.HIGHEST)
            nbr.append(g)
        if j < steps - 1:
            d2 = jnp.where(oh, inf, d2)

    p_locs = []
    cats = []
    for k in range(K):
        g = nbr[k]
        p_loc = g[:, :3] - rep
        f_nb = g[:, 3:]
        h = _elu(jnp.dot(p_loc, W1, preferred_element_type=jnp.float32, precision=jax.lax.Precision.HIGHEST) + b1)
        h = _elu(jnp.dot(h, W2, preferred_element_type=jnp.float32, precision=jax.lax.Precision.HIGHEST) + b2)
        p_locs.append(p_loc)
        cats.append(jnp.concatenate([h, f_nb], axis=1))

    Xin = jnp.concatenate(p_locs, axis=1)
    X = _elu(jnp.dot(Xin, Wx0, preferred_element_type=jnp.float32, precision=jax.lax.Precision.HIGHEST) + bx0)
    X = _elu(jnp.dot(X, Wx1, preferred_element_type=jnp.float32, precision=jax.lax.Precision.HIGHEST) + bx1)
    X = jnp.dot(X, Wx2, preferred_element_type=jnp.float32, precision=jax.lax.Precision.HIGHEST) + bx2

    ftsX = []
    for k in range(K):
        acc = X[:, k * K:k * K + 1] * cats[0]
        for j in range(1, K):
            acc = acc + X[:, k * K + j:k * K + j + 1] * cats[j]
        ftsX.append(acc)

    cols = []
    for d in range(2):
        c = ftsX[0] * Wdt[d * K:d * K + 1, :]
        for k in range(1, K):
            c = c + ftsX[k] * Wdt[d * K + k:d * K + k + 1, :]
        cols.append(c)
    dw = jnp.concatenate(cols, axis=1) + bdp

    out_ref[0] = _elu(jnp.dot(dw, Wpp, preferred_element_type=jnp.float32, precision=jax.lax.Precision.HIGHEST)
                      + bp)


def _prep_weights(p, K, Cp, Cout):
    Wd = p['Wd']
    Wdt = jnp.transpose(Wd, (1, 2, 0)).reshape(2 * K, Cp)
    bdp = p['bd'].reshape(Cp, 2).T.reshape(1, 2 * Cp)
    Wpp = jnp.transpose(p['Wp'].reshape(Cp, 2, Cout), (1, 0, 2))
    Wpp = Wpp.reshape(2 * Cp, Cout)
    return (p['Wl'], p['bl'].reshape(1, -1),
            p['W1'], p['b1'].reshape(1, -1),
            p['W2'], p['b2'].reshape(1, -1),
            p['Wx0'], p['bx0'].reshape(1, -1),
            p['Wx1'], p['bx1'].reshape(1, -1),
            p['Wx2'], p['bx2'].reshape(1, -1),
            Wdt, bdp, Wpp, p['bp'].reshape(1, -1))


def _xconv_layer(pts, fts, p, cfg, Pb, seloh=None):
    Cin, Cout, K, D, _ = cfg
    B, N, _ = pts.shape
    lift = Cout // 2
    Cmid = Cout // 4
    Cp = Cmid + lift
    steps = (K - 1) * D + 2
    sel = seloh is not None
    P = seloh.shape[0] if sel else N
    nblk = P // Pb

    weights = _prep_weights(p, K, Cp, Cout)
    wspecs = [pl.BlockSpec(w.shape, lambda b, pb: (0,) * w.ndim)
              for w in weights]

    in_specs = [
        pl.BlockSpec((1, N, 3), lambda b, pb: (b, 0, 0)),
        pl.BlockSpec((1, 3, N), lambda b, pb: (b, 0, 0)),
        pl.BlockSpec((1, N, Cin), lambda b, pb: (b, 0, 0)),
    ]
    inputs = [pts, jnp.transpose(pts, (0, 2, 1)), fts]
    if sel:
        in_specs.append(pl.BlockSpec((P, N), lambda b, pb: (0, 0)))
        inputs.append(seloh)
        out_shape = [
            jax.ShapeDtypeStruct((B, P, Cout), jnp.float32),
            jax.ShapeDtypeStruct((B, P, 3), jnp.float32),
        ]
        out_specs = [
            pl.BlockSpec((1, P, Cout), lambda b, pb: (b, 0, 0)),
            pl.BlockSpec((1, P, 3), lambda b, pb: (b, 0, 0)),
        ]
    else:
        in_specs.append(pl.BlockSpec((1, Pb, 3), lambda b, pb: (b, pb, 0)))
        inputs.append(pts)
        out_shape = jax.ShapeDtypeStruct((B, P, Cout), jnp.float32)
        out_specs = pl.BlockSpec((1, Pb, Cout), lambda b, pb: (b, pb, 0))
    in_specs.extend(wspecs)
    inputs.extend(weights)

    body = functools.partial(_xconv_body, N=N, Pb=(P if sel else Pb),
                             K=K, D=D, steps=steps, sel=sel)
    return pl.pallas_call(
        body,
        grid=(B, nblk),
        in_specs=in_specs,
        out_specs=out_specs,
        out_shape=out_shape,
        interpret=_INTERPRET,
    )(*inputs)


def _head_body(fts_ref, W0, b0, W1, b1, W2, b2, out_ref):
    f = fts_ref[0]
    h = _elu(jnp.dot(f, W0[...], preferred_element_type=jnp.float32, precision=jax.lax.Precision.HIGHEST)
             + b0[...])
    h = _elu(jnp.dot(h, W1[...], preferred_element_type=jnp.float32, precision=jax.lax.Precision.HIGHEST)
             + b1[...])
    lg = jnp.dot(h, W2[...], preferred_element_type=jnp.float32, precision=jax.lax.Precision.HIGHEST) + b2[...]
    out_ref[0] = jnp.mean(lg, axis=0, keepdims=True)


def _head(fts, params):
    B, P, C = fts.shape
    ws = (params['fc0_W'], params['fc0_b'].reshape(1, -1),
          params['fc1_W'], params['fc1_b'].reshape(1, -1),
          params['fc2_W'], params['fc2_b'].reshape(1, -1))
    wspecs = [pl.BlockSpec(w.shape, lambda b: (0,) * w.ndim) for w in ws]
    NC = params['fc2_W'].shape[1]
    out = pl.pallas_call(
        _head_body,
        grid=(B,),
        in_specs=[pl.BlockSpec((1, P, C), lambda b: (b, 0, 0))] + wspecs,
        out_specs=pl.BlockSpec((1, 1, NC), lambda b: (b, 0, 0)),
        out_shape=jax.ShapeDtypeStruct((B, 1, NC), jnp.float32),
        interpret=_INTERPRET,
    )(fts, *ws)
    return out.reshape(B, NC)


def kernel(x, params):
    B, N, _ = x.shape

    key = jax.random.key(42)
    subs = []
    for _ in range(len(_LAYER_CFG)):
        key, sub = jax.random.split(key)
        subs.append(sub)
    sel = jax.random.permutation(subs[3], N)[:_LAYER_CFG[3][4]]
    seloh = jax.nn.one_hot(sel, N, dtype=jnp.float32)

    pts, fts = x, x
    layers = params['layers']
    fts = _xconv_layer(pts, fts, layers[0], _LAYER_CFG[0], Pb=256)
    fts = _xconv_layer(pts, fts, layers[1], _LAYER_CFG[1], Pb=256)
    fts = _xconv_layer(pts, fts, layers[2], _LAYER_CFG[2], Pb=256)
    fts, rep = _xconv_layer(pts, fts, layers[3], _LAYER_CFG[3], Pb=120,
                            seloh=seloh)
    fts = _xconv_layer(rep, fts, layers[4], _LAYER_CFG[4], Pb=120)
    return _head(fts, params)


# speedup vs baseline: 7.3579x; 1.8647x over previous
import functools

import jax
import jax.numpy as jnp
from jax.experimental import pallas as pl

_INTERPRET = False

_LAYER_CFG = [
    (3, 32, 8, 1, -1),
    (32, 64, 8, 2, -1),
    (64, 96, 8, 4, -1),
    (96, 128, 12, 4, 120),
    (128, 160, 12, 6, 120),
]


def _elu(x):
    return jnp.where(x > 0, x, jnp.exp(jnp.minimum(x, 0.0)) - 1.0)


def _prelift_body(pts_ref, fts_ref, Wl, bl, g_ref):
    lifted = _elu(jnp.dot(fts_ref[0], Wl[...],
                          preferred_element_type=jnp.float32) + bl[...])
    g_ref[0] = jnp.concatenate([pts_ref[0], lifted], axis=1)


def _xconv_body(g_ref, ptsT_ref, *rest, N, Pb, K, D, steps, sel):
    if sel:
        seloh_ref = rest[0]
        wrefs = rest[1:15]
        out_ref = rest[15]
        rep_out_ref = rest[16]
    else:
        rep_ref = rest[0]
        wrefs = rest[1:15]
        out_ref = rest[15]
    (W1, b1, W2, b2, Wx0, bx0, Wx1, bx1, Wx2, bx2,
     Wdt, bdp, Wpp, bp) = [w[...] for w in wrefs]

    G = g_ref[0]
    CG = G.shape[1]

    s1 = G.astype(jnp.bfloat16)
    r = G - s1.astype(jnp.float32)
    s2 = r.astype(jnp.bfloat16)
    r = r - s2.astype(jnp.float32)
    G3 = jnp.concatenate([s1, s2, r.astype(jnp.bfloat16)], axis=1)

    def exact_gather(onehot):
        t = jnp.dot(onehot.astype(jnp.bfloat16), G3,
                    preferred_element_type=jnp.float32)
        return t[:, :CG] + t[:, CG:2 * CG] + t[:, 2 * CG:]

    if sel:
        rep = exact_gather(seloh_ref[...])[:, :3]
        rep_out_ref[0] = rep
    else:
        rep = rep_ref[0]

    ptsT = ptsT_ref[0]
    d2 = None
    for c in range(3):
        diff = rep[:, c:c + 1] - ptsT[c:c + 1, :]
        sq = diff * diff
        d2 = sq if d2 is None else d2 + sq

    iota = jax.lax.broadcasted_iota(jnp.int32, (Pb, N), 1)
    inf = jnp.float32(jnp.inf)

    d2 = jnp.where(d2 <= 0.0, inf, d2)

    nbr = []
    for j in range(1, steps):
        m = jnp.min(d2, axis=1, keepdims=True)
        if (j - 1) % D == 0 and len(nbr) < K:
            cand = jnp.where(d2 <= m, iota, N)
            idx = jnp.min(cand, axis=1, keepdims=True)
            oh = iota == idx
            nbr.append(exact_gather(oh))
            if j < steps - 1:
                d2 = jnp.where(oh, inf, d2)
        else:
            d2 = jnp.where(d2 <= m, inf, d2)

    p_locs = []
    cats = []
    for k in range(K):
        g = nbr[k]
        p_loc = g[:, :3] - rep
        f_nb = g[:, 3:]
        h = _elu(jnp.dot(p_loc, W1, preferred_element_type=jnp.float32) + b1)
        h = _elu(jnp.dot(h, W2, preferred_element_type=jnp.float32) + b2)
        p_locs.append(p_loc)
        cats.append(jnp.concatenate([h, f_nb], axis=1))

    Xin = jnp.concatenate(p_locs, axis=1)
    X = _elu(jnp.dot(Xin, Wx0, preferred_element_type=jnp.float32) + bx0)
    X = _elu(jnp.dot(X, Wx1, preferred_element_type=jnp.float32) + bx1)
    X = jnp.dot(X, Wx2, preferred_element_type=jnp.float32) + bx2

    ftsX = []
    for k in range(K):
        acc = X[:, k * K:k * K + 1] * cats[0]
        for j in range(1, K):
            acc = acc + X[:, k * K + j:k * K + j + 1] * cats[j]
        ftsX.append(acc)

    cols = []
    for d in range(2):
        c = ftsX[0] * Wdt[d * K:d * K + 1, :]
        for k in range(1, K):
            c = c + ftsX[k] * Wdt[d * K + k:d * K + k + 1, :]
        cols.append(c)
    dw = jnp.concatenate(cols, axis=1) + bdp

    out_ref[0] = _elu(jnp.dot(dw, Wpp, preferred_element_type=jnp.float32)
                      + bp)


def _prep_weights(p, K, Cp, Cout):
    Wd = p['Wd']
    Wdt = jnp.transpose(Wd, (1, 2, 0)).reshape(2 * K, Cp)
    bdp = p['bd'].reshape(Cp, 2).T.reshape(1, 2 * Cp)
    Wpp = jnp.transpose(p['Wp'].reshape(Cp, 2, Cout), (1, 0, 2))
    Wpp = Wpp.reshape(2 * Cp, Cout)
    return (p['Wl'], p['bl'].reshape(1, -1),
            p['W1'], p['b1'].reshape(1, -1),
            p['W2'], p['b2'].reshape(1, -1),
            p['Wx0'], p['bx0'].reshape(1, -1),
            p['Wx1'], p['bx1'].reshape(1, -1),
            p['Wx2'], p['bx2'].reshape(1, -1),
            Wdt, bdp, Wpp, p['bp'].reshape(1, -1))


def _xconv_layer(pts, fts, p, cfg, Pb, seloh=None):
    Cin, Cout, K, D, _ = cfg
    B, N, _ = pts.shape
    lift = Cout // 2
    Cmid = Cout // 4
    Cp = Cmid + lift
    steps = (K - 1) * D + 2
    sel = seloh is not None
    P = seloh.shape[0] if sel else N
    nblk = P // Pb

    (Wl, bl, W1, b1, W2, b2, Wx0, bx0, Wx1, bx1, Wx2, bx2,
     Wdt, bdp, Wpp, bp) = _prep_weights(p, K, Cp, Cout)

    CG = 3 + lift
    G = pl.pallas_call(
        _prelift_body,
        grid=(B,),
        in_specs=[
            pl.BlockSpec((1, N, 3), lambda b: (b, 0, 0)),
            pl.BlockSpec((1, N, Cin), lambda b: (b, 0, 0)),
            pl.BlockSpec(Wl.shape, lambda b: (0, 0)),
            pl.BlockSpec(bl.shape, lambda b: (0, 0)),
        ],
        out_specs=pl.BlockSpec((1, N, CG), lambda b: (b, 0, 0)),
        out_shape=jax.ShapeDtypeStruct((B, N, CG), jnp.float32),
        interpret=_INTERPRET,
    )(pts, fts, Wl, bl)

    weights = (W1, b1, W2, b2, Wx0, bx0, Wx1, bx1, Wx2, bx2,
               Wdt, bdp, Wpp, bp)
    wspecs = [pl.BlockSpec(w.shape, lambda b, pb: (0, 0)) for w in weights]

    in_specs = [
        pl.BlockSpec((1, N, CG), lambda b, pb: (b, 0, 0)),
        pl.BlockSpec((1, 3, N), lambda b, pb: (b, 0, 0)),
    ]
    inputs = [G, jnp.transpose(pts, (0, 2, 1))]
    if sel:
        in_specs.append(pl.BlockSpec((P, N), lambda b, pb: (0, 0)))
        inputs.append(seloh)
        out_shape = [
            jax.ShapeDtypeStruct((B, P, Cout), jnp.float32),
            jax.ShapeDtypeStruct((B, P, 3), jnp.float32),
        ]
        out_specs = [
            pl.BlockSpec((1, P, Cout), lambda b, pb: (b, 0, 0)),
            pl.BlockSpec((1, P, 3), lambda b, pb: (b, 0, 0)),
        ]
    else:
        in_specs.append(pl.BlockSpec((1, Pb, 3), lambda b, pb: (b, pb, 0)))
        inputs.append(pts)
        out_shape = jax.ShapeDtypeStruct((B, P, Cout), jnp.float32)
        out_specs = pl.BlockSpec((1, Pb, Cout), lambda b, pb: (b, pb, 0))
    in_specs.extend(wspecs)
    inputs.extend(weights)

    body = functools.partial(_xconv_body, N=N, Pb=(P if sel else Pb),
                             K=K, D=D, steps=steps, sel=sel)
    return pl.pallas_call(
        body,
        grid=(B, nblk),
        in_specs=in_specs,
        out_specs=out_specs,
        out_shape=out_shape,
        interpret=_INTERPRET,
    )(*inputs)


def _head_body(fts_ref, W0, b0, W1, b1, W2, b2, out_ref):
    f = fts_ref[0]
    h = _elu(jnp.dot(f, W0[...], preferred_element_type=jnp.float32)
             + b0[...])
    h = _elu(jnp.dot(h, W1[...], preferred_element_type=jnp.float32)
             + b1[...])
    lg = jnp.dot(h, W2[...], preferred_element_type=jnp.float32) + b2[...]
    out_ref[0] = jnp.mean(lg, axis=0, keepdims=True)


def _head(fts, params):
    B, P, C = fts.shape
    ws = (params['fc0_W'], params['fc0_b'].reshape(1, -1),
          params['fc1_W'], params['fc1_b'].reshape(1, -1),
          params['fc2_W'], params['fc2_b'].reshape(1, -1))
    wspecs = [pl.BlockSpec(w.shape, lambda b: (0,) * w.ndim) for w in ws]
    NC = params['fc2_W'].shape[1]
    out = pl.pallas_call(
        _head_body,
        grid=(B,),
        in_specs=[pl.BlockSpec((1, P, C), lambda b: (b, 0, 0))] + wspecs,
        out_specs=pl.BlockSpec((1, 1, NC), lambda b: (b, 0, 0)),
        out_shape=jax.ShapeDtypeStruct((B, 1, NC), jnp.float32),
        interpret=_INTERPRET,
    )(fts, *ws)
    return out.reshape(B, NC)


def kernel(x, params):
    B, N, _ = x.shape

    key = jax.random.key(42)
    subs = []
    for _ in range(len(_LAYER_CFG)):
        key, sub = jax.random.split(key)
        subs.append(sub)
    sel = jax.random.permutation(subs[3], N)[:_LAYER_CFG[3][4]]
    seloh = jax.nn.one_hot(sel, N, dtype=jnp.float32)

    pts, fts = x, x
    layers = params['layers']
    fts = _xconv_layer(pts, fts, layers[0], _LAYER_CFG[0], Pb=256)
    fts = _xconv_layer(pts, fts, layers[1], _LAYER_CFG[1], Pb=256)
    fts = _xconv_layer(pts, fts, layers[2], _LAYER_CFG[2], Pb=256)
    fts, rep = _xconv_layer(pts, fts, layers[3], _LAYER_CFG[3], Pb=120,
                            seloh=seloh)
    fts = _xconv_layer(rep, fts, layers[4], _LAYER_CFG[4], Pb=120)
    return _head(fts, params)


# speedup vs baseline: 10.5686x; 1.4364x over previous
import functools

import jax
import jax.numpy as jnp
from jax.experimental import pallas as pl

_INTERPRET = False

_LAYER_CFG = [
    (3, 32, 8, 1, -1),
    (32, 64, 8, 2, -1),
    (64, 96, 8, 4, -1),
    (96, 128, 12, 4, 120),
    (128, 160, 12, 6, 120),
]


def _elu(x):
    return jnp.where(x > 0, x, jnp.exp(jnp.minimum(x, 0.0)) - 1.0)


def _prelift_body(pts_ref, fts_ref, Wl, bl, g_ref):
    lifted = _elu(jnp.dot(fts_ref[0], Wl[...],
                          preferred_element_type=jnp.float32) + bl[...])
    g_ref[0] = jnp.concatenate([pts_ref[0], lifted], axis=1)


def _xconv_body(g_ref, ptsT_ref, *rest, N, Pb, K, D, steps, sel):
    if sel:
        seloh_ref = rest[0]
        wrefs = rest[1:15]
        out_ref = rest[15]
        rep_out_ref = rest[16]
    else:
        rep_ref = rest[0]
        wrefs = rest[1:15]
        out_ref = rest[15]
    (W1, b1, W2, b2, Wx0, bx0, Wx1, bx1, Wx2, bx2,
     Wbig, bdp, Wpp, bp) = [w[...] for w in wrefs]

    G = g_ref[0]
    CG = G.shape[1]

    s1 = G.astype(jnp.bfloat16)
    r = G - s1.astype(jnp.float32)
    s2 = r.astype(jnp.bfloat16)
    r = r - s2.astype(jnp.float32)
    G3 = jnp.concatenate([s1, s2, r.astype(jnp.bfloat16)], axis=1)

    def exact_gather(onehot):
        t = jnp.dot(onehot.astype(jnp.bfloat16), G3,
                    preferred_element_type=jnp.float32)
        return t[:, :CG] + t[:, CG:2 * CG] + t[:, 2 * CG:]

    if sel:
        rep = exact_gather(seloh_ref[...])[:, :3]
        rep_out_ref[0] = rep
    else:
        rep = rep_ref[0]

    ptsT = ptsT_ref[0]
    d2 = None
    for c in range(3):
        diff = rep[:, c:c + 1] - ptsT[c:c + 1, :]
        sq = diff * diff
        d2 = sq if d2 is None else d2 + sq

    iota = jax.lax.broadcasted_iota(
        jnp.int32, (Pb, N), 1).astype(jnp.float32)
    inf = jnp.float32(jnp.inf)
    nf = jnp.float32(N)

    d2 = jnp.where(d2 <= 0.0, inf, d2)

    nbr = []
    for j in range(1, steps):
        m = jnp.min(d2, axis=1, keepdims=True)
        if (j - 1) % D == 0 and len(nbr) < K:
            cand = jnp.where(d2 <= m, iota, nf)
            idx = jnp.min(cand, axis=1, keepdims=True)
            oh = iota == idx
            nbr.append(exact_gather(oh))
            if j < steps - 1:
                d2 = jnp.where(oh, inf, d2)
        else:
            d2 = jnp.where(d2 <= m, inf, d2)

    p_locs = []
    cats = []
    for k in range(K):
        g = nbr[k]
        p_loc = g[:, :3] - rep
        f_nb = g[:, 3:]
        h = _elu(jnp.dot(p_loc, W1, preferred_element_type=jnp.float32) + b1)
        h = _elu(jnp.dot(h, W2, preferred_element_type=jnp.float32) + b2)
        p_locs.append(p_loc)
        cats.append(jnp.concatenate([h, f_nb], axis=1))

    Xin = jnp.concatenate(p_locs, axis=1)
    X = _elu(jnp.dot(Xin, Wx0, preferred_element_type=jnp.float32) + bx0)
    X = _elu(jnp.dot(X, Wx1, preferred_element_type=jnp.float32) + bx1)
    X = jnp.dot(X, Wx2, preferred_element_type=jnp.float32) + bx2

    Cp = cats[0].shape[1]
    C_all = jnp.dot(X, Wbig, preferred_element_type=jnp.float32)
    cols = []
    for d in range(2):
        acc = None
        for j in range(K):
            base = (j * 2 + d) * 128
            t = cats[j] * C_all[:, base:base + Cp]
            acc = t if acc is None else acc + t
        cols.append(acc)
    dw = jnp.concatenate(cols, axis=1) + bdp

    out_ref[0] = _elu(jnp.dot(dw, Wpp, preferred_element_type=jnp.float32)
                      + bp)


def _prep_weights(p, K, Cp, Cout):
    Wd = p['Wd']
    bdp = p['bd'].reshape(Cp, 2).T.reshape(1, 2 * Cp)
    Wpp = jnp.transpose(p['Wp'].reshape(Cp, 2, Cout), (1, 0, 2))
    Wpp = Wpp.reshape(2 * Cp, Cout)
    perm = jnp.arange(K * K).reshape(K, K).T.reshape(-1)
    Wx2p = p['Wx2'][:, perm]
    bx2p = p['bx2'][perm].reshape(1, -1)
    base = jnp.pad(jnp.transpose(Wd, (2, 1, 0)),
                   ((0, 0), (0, 0), (0, 128 - Cp)))
    eye = jnp.eye(K, dtype=jnp.float32)
    Wbig = (eye[:, None, :, None, None]
            * base[None, :, None, :, :]).reshape(K * K, K * 2 * 128)
    return (p['Wl'], p['bl'].reshape(1, -1),
            p['W1'], p['b1'].reshape(1, -1),
            p['W2'], p['b2'].reshape(1, -1),
            p['Wx0'], p['bx0'].reshape(1, -1),
            p['Wx1'], p['bx1'].reshape(1, -1),
            Wx2p, bx2p,
            Wbig, bdp, Wpp, p['bp'].reshape(1, -1))


def _xconv_layer(pts, fts, p, cfg, Pb, seloh=None):
    Cin, Cout, K, D, _ = cfg
    B, N, _ = pts.shape
    lift = Cout // 2
    Cmid = Cout // 4
    Cp = Cmid + lift
    steps = (K - 1) * D + 2
    sel = seloh is not None
    P = seloh.shape[0] if sel else N
    nblk = P // Pb

    (Wl, bl, W1, b1, W2, b2, Wx0, bx0, Wx1, bx1, Wx2, bx2,
     Wdt, bdp, Wpp, bp) = _prep_weights(p, K, Cp, Cout)

    CG = 3 + lift
    G = pl.pallas_call(
        _prelift_body,
        grid=(B,),
        in_specs=[
            pl.BlockSpec((1, N, 3), lambda b: (b, 0, 0)),
            pl.BlockSpec((1, N, Cin), lambda b: (b, 0, 0)),
            pl.BlockSpec(Wl.shape, lambda b: (0, 0)),
            pl.BlockSpec(bl.shape, lambda b: (0, 0)),
        ],
        out_specs=pl.BlockSpec((1, N, CG), lambda b: (b, 0, 0)),
        out_shape=jax.ShapeDtypeStruct((B, N, CG), jnp.float32),
        interpret=_INTERPRET,
    )(pts, fts, Wl, bl)

    weights = (W1, b1, W2, b2, Wx0, bx0, Wx1, bx1, Wx2, bx2,
               Wdt, bdp, Wpp, bp)
    wspecs = [pl.BlockSpec(w.shape, lambda b, pb: (0, 0)) for w in weights]

    in_specs = [
        pl.BlockSpec((1, N, CG), lambda b, pb: (b, 0, 0)),
        pl.BlockSpec((1, 3, N), lambda b, pb: (b, 0, 0)),
    ]
    inputs = [G, jnp.transpose(pts, (0, 2, 1))]
    if sel:
        in_specs.append(pl.BlockSpec((P, N), lambda b, pb: (0, 0)))
        inputs.append(seloh)
        out_shape = [
            jax.ShapeDtypeStruct((B, P, Cout), jnp.float32),
            jax.ShapeDtypeStruct((B, P, 3), jnp.float32),
        ]
        out_specs = [
            pl.BlockSpec((1, P, Cout), lambda b, pb: (b, 0, 0)),
            pl.BlockSpec((1, P, 3), lambda b, pb: (b, 0, 0)),
        ]
    else:
        in_specs.append(pl.BlockSpec((1, Pb, 3), lambda b, pb: (b, pb, 0)))
        inputs.append(pts)
        out_shape = jax.ShapeDtypeStruct((B, P, Cout), jnp.float32)
        out_specs = pl.BlockSpec((1, Pb, Cout), lambda b, pb: (b, pb, 0))
    in_specs.extend(wspecs)
    inputs.extend(weights)

    body = functools.partial(_xconv_body, N=N, Pb=(P if sel else Pb),
                             K=K, D=D, steps=steps, sel=sel)
    return pl.pallas_call(
        body,
        grid=(B, nblk),
        in_specs=in_specs,
        out_specs=out_specs,
        out_shape=out_shape,
        interpret=_INTERPRET,
    )(*inputs)


def _head_body(fts_ref, W0, b0, W1, b1, W2, b2, out_ref):
    f = fts_ref[0]
    h = _elu(jnp.dot(f, W0[...], preferred_element_type=jnp.float32)
             + b0[...])
    h = _elu(jnp.dot(h, W1[...], preferred_element_type=jnp.float32)
             + b1[...])
    lg = jnp.dot(h, W2[...], preferred_element_type=jnp.float32) + b2[...]
    out_ref[0] = jnp.mean(lg, axis=0, keepdims=True)


def _head(fts, params):
    B, P, C = fts.shape
    ws = (params['fc0_W'], params['fc0_b'].reshape(1, -1),
          params['fc1_W'], params['fc1_b'].reshape(1, -1),
          params['fc2_W'], params['fc2_b'].reshape(1, -1))
    wspecs = [pl.BlockSpec(w.shape, lambda b: (0,) * w.ndim) for w in ws]
    NC = params['fc2_W'].shape[1]
    out = pl.pallas_call(
        _head_body,
        grid=(B,),
        in_specs=[pl.BlockSpec((1, P, C), lambda b: (b, 0, 0))] + wspecs,
        out_specs=pl.BlockSpec((1, 1, NC), lambda b: (b, 0, 0)),
        out_shape=jax.ShapeDtypeStruct((B, 1, NC), jnp.float32),
        interpret=_INTERPRET,
    )(fts, *ws)
    return out.reshape(B, NC)


def kernel(x, params):
    B, N, _ = x.shape

    key = jax.random.key(42)
    subs = []
    for _ in range(len(_LAYER_CFG)):
        key, sub = jax.random.split(key)
        subs.append(sub)
    sel = jax.random.permutation(subs[3], N)[:_LAYER_CFG[3][4]]
    seloh = jax.nn.one_hot(sel, N, dtype=jnp.float32)

    pts, fts = x, x
    layers = params['layers']
    fts = _xconv_layer(pts, fts, layers[0], _LAYER_CFG[0], Pb=256)
    fts = _xconv_layer(pts, fts, layers[1], _LAYER_CFG[1], Pb=256)
    fts = _xconv_layer(pts, fts, layers[2], _LAYER_CFG[2], Pb=256)
    fts, rep = _xconv_layer(pts, fts, layers[3], _LAYER_CFG[3], Pb=120,
                            seloh=seloh)
    fts = _xconv_layer(rep, fts, layers[4], _LAYER_CFG[4], Pb=120)
    return _head(fts, params)
